# baseline (speedup 1.0000x reference)
import jax
import jax.numpy as jnp
from jax import lax
from jax.experimental import pallas as pl
from jax.experimental.pallas import tpu as pltpu
from jax.experimental.pallas import tpu_sc as plsc

N_CHANNELS = 384
N_NEIGHBORHOODS = 512
NHOOD_SIZE = 40
N_SPIKES = 1000000
N_FEAT = 8
MIN_COVERAGE = 0.12

NW = 32
BLK = 128
NBLK = N_SPIKES // BLK
TAIL = N_SPIKES - NBLK * BLK
BPW = NBLK // NW
EXTRA_B = NBLK - BPW * NW
KBUF = 8
OUTER = (BPW + KBUF - 1) // KBUF
NB_PER_W = N_NEIGHBORHOODS // NW


def _sc_kernel(ids_hbm, featT_hbm, nbf_hbm, chan_hbm, tailf_hbm,
               covn_hbm, ccnt_hbm, parts_hbm, pops_hbm,
               ids3, feat3, a0, a1, a2, a3, a4, a5, a6, a7,
               pop_v, ind_v, nb_v, chan_v, cov_st, cnt_st, tailf_v,
               s0, s1, s2, s3, s4, s5, s6, s7):
    w = lax.axis_index("c") * 16 + lax.axis_index("s")
    iota = jnp.arange(16, dtype=jnp.int32)
    zeros_f = jnp.zeros((16,), jnp.float32)
    ones_f = jnp.full((16,), 1.0, jnp.float32)
    ones_i = jnp.full((16,), 1, jnp.int32)
    zeros_i = jnp.zeros((16,), jnp.int32)
    accs = [a0, a1, a2, a3, a4, a5, a6, a7]

    pltpu.sync_copy(nbf_hbm.at[pl.ds(w * (NB_PER_W * NHOOD_SIZE),
                                     NB_PER_W * NHOOD_SIZE)], nb_v)
    pltpu.sync_copy(chan_hbm, chan_v)

    def _zero_ind(k, carry):
        ind_v[pl.ds(k * 16, 16)] = zeros_f
        return carry
    lax.fori_loop(0, N_CHANNELS // 16, _zero_ind, 0)

    ch0 = chan_v[pl.ds(0, 16)]
    ch1 = chan_v[pl.ds(16, 16)]
    ch2 = chan_v[pl.ds(32, 16)]

    def _nb_body(jj, carry):
        num_vec, cnt_vec = carry
        b = jj * NHOOD_SIZE
        v0 = nb_v[pl.ds(b, 16)]
        v1 = nb_v[pl.ds(b + 16, 16)]
        v2 = nb_v[pl.ds(b + 24, 16)]
        plsc.store_scatter(ind_v, [v0], ones_f)
        plsc.store_scatter(ind_v, [v1], ones_f)
        plsc.store_scatter(ind_v, [v2], ones_f)

        def _sum_ind(k, s):
            return s + ind_v[pl.ds(k * 16, 16)]
        cnt = jnp.sum(lax.fori_loop(0, N_CHANNELS // 16, _sum_ind, zeros_f))
        hits = (plsc.load_gather(ind_v, [ch0])
                + plsc.load_gather(ind_v, [ch1])
                + plsc.load_gather(ind_v, [ch2]))
        here = iota == jj
        num_vec = jnp.where(here, jnp.sum(hits), num_vec)
        cnt_vec = jnp.where(here, cnt, cnt_vec)
        plsc.store_scatter(ind_v, [v0], zeros_f)
        plsc.store_scatter(ind_v, [v1], zeros_f)
        plsc.store_scatter(ind_v, [v2], zeros_f)
        return num_vec, cnt_vec
    num_vec, cnt_vec = lax.fori_loop(
        0, NB_PER_W, _nb_body, (zeros_f, zeros_f))
    cov_st[...] = num_vec
    cnt_st[...] = cnt_vec

    pltpu.sync_copy(cov_st, covn_hbm.at[pl.ds(w * NB_PER_W, NB_PER_W)])
    pltpu.sync_copy(cnt_st, ccnt_hbm.at[pl.ds(w * NB_PER_W, NB_PER_W)])

    def _zero_acc(k, carry):
        for a in accs:
            a[pl.ds(k * 16, 16)] = zeros_f
        pop_v[pl.ds(k * 16, 16)] = zeros_i
        return carry
    lax.fori_loop(0, N_NEIGHBORHOODS // 16, _zero_acc, 0)

    sems = [s0, s1, s2, s3, s4, s5, s6, s7]
    base_b = w * BPW

    def _fire(b, k):
        off = (base_b + b) * BLK
        pltpu.async_copy(ids_hbm.at[pl.ds(off, BLK)], ids3.at[k], sems[k])
        pltpu.async_copy(featT_hbm.at[:, pl.ds(off, BLK)], feat3.at[k], sems[k])

    def _wait(k):
        pltpu.make_async_copy(ids_hbm.at[pl.ds(0, BLK)],
                              ids3.at[k], sems[k]).wait()
        pltpu.make_async_copy(featT_hbm.at[:, pl.ds(0, BLK)],
                              feat3.at[k], sems[k]).wait()

    def _process_block(k):
        for g in range(BLK // 16):
            s = g * 16
            ids16 = ids3[k, pl.ds(s, 16)]
            plsc.addupdate_scatter(pop_v, [ids16], ones_i)
            for j in range(N_FEAT):
                x = feat3[k, j, pl.ds(s, 16)]
                plsc.addupdate_scatter(accs[j], [ids16], x)

    for k in range(KBUF):
        _fire(k, k)

    def _round(o, carry):
        for k in range(KBUF):
            b = o * KBUF + k

            @pl.when(b < BPW)
            def _do():
                _wait(k)
                _process_block(k)

                @pl.when(b + KBUF < BPW)
                def _refire():
                    _fire(b + KBUF, k)
        return carry
    lax.fori_loop(0, OUTER, _round, 0)

    @pl.when(w < EXTRA_B)
    def _extra():
        off = NW * (BPW * BLK) + w * BLK
        pltpu.sync_copy(ids_hbm.at[pl.ds(off, BLK)], ids3.at[0])
        pltpu.sync_copy(featT_hbm.at[:, pl.ds(off, BLK)], feat3.at[0])
        _process_block(0)

    @pl.when(w == NW - 1)
    def _tail():
        off = NBLK * BLK
        pltpu.sync_copy(ids_hbm.at[pl.ds(off, TAIL)],
                        ids3.at[0].at[pl.ds(0, TAIL)])
        pltpu.sync_copy(tailf_hbm, tailf_v)

        def _tgroup(t, carry):
            ids16 = ids3[0, pl.ds(t * 16, 16)]
            plsc.addupdate_scatter(pop_v, [ids16], ones_i)
            for j in range(N_FEAT):
                x = tailf_v[pl.ds(j * TAIL + t * 16, 16)]
                plsc.addupdate_scatter(accs[j], [ids16], x)
            return carry
        lax.fori_loop(0, TAIL // 16, _tgroup, 0)

    for j in range(N_FEAT):
        pltpu.sync_copy(
            accs[j],
            parts_hbm.at[pl.ds(w * (N_FEAT * N_NEIGHBORHOODS)
                               + j * N_NEIGHBORHOODS, N_NEIGHBORHOODS)])
    pltpu.sync_copy(pop_v, pops_hbm.at[pl.ds(w * N_NEIGHBORHOODS,
                                             N_NEIGHBORHOODS)])


def _combine_body(parts_ref, pops_ref, covn_ref, ccnt_ref,
                  cov_ref, covd_ref, means_ref, ns_ref):
    s = parts_ref[0]
    for wq in range(1, NW):
        s = s + parts_ref[wq]
    pop = jnp.sum(pops_ref[...], axis=0)
    denom = jnp.maximum(pop, 1).astype(jnp.float32)
    means_ref[...] = s.T / denom[:, None]
    coverage = covn_ref[...] / ccnt_ref[...]
    cov_ref[...] = coverage
    covered = coverage >= MIN_COVERAGE
    covd_ref[...] = covered.astype(jnp.int32)
    ns_ref[0, 0] = jnp.sum(jnp.where(covered, pop, 0))


@jax.jit
def kernel(neighborhood_ids, neighborhoods, channels, features):
    ids = neighborhood_ids.astype(jnp.int32)
    nbf = neighborhoods.astype(jnp.int32).reshape(-1)
    chan = channels.astype(jnp.int32)
    featT = features.astype(jnp.float32).T
    tailf = featT[:, NBLK * BLK:].reshape(-1)

    mesh = plsc.VectorSubcoreMesh(core_axis_name="c", subcore_axis_name="s")
    sc = pl.kernel(
        _sc_kernel,
        out_type=(
            jax.ShapeDtypeStruct((N_NEIGHBORHOODS,), jnp.float32),
            jax.ShapeDtypeStruct((N_NEIGHBORHOODS,), jnp.float32),
            jax.ShapeDtypeStruct((NW * N_FEAT * N_NEIGHBORHOODS,), jnp.float32),
            jax.ShapeDtypeStruct((NW * N_NEIGHBORHOODS,), jnp.int32),
        ),
        mesh=mesh,
        compiler_params=pltpu.CompilerParams(needs_layout_passes=False,
                                             use_tc_tiling_on_sc=True),
        scratch_types=[
            pltpu.VMEM((KBUF, BLK), jnp.int32),
            pltpu.VMEM((KBUF, N_FEAT, BLK), jnp.float32),
        ] + [pltpu.VMEM((N_NEIGHBORHOODS,), jnp.float32)] * N_FEAT + [
            pltpu.VMEM((N_NEIGHBORHOODS,), jnp.int32),
            pltpu.VMEM((N_CHANNELS,), jnp.float32),
            pltpu.VMEM((NB_PER_W * NHOOD_SIZE,), jnp.int32),
            pltpu.VMEM((48,), jnp.int32),
            pltpu.VMEM((NB_PER_W,), jnp.float32),
            pltpu.VMEM((NB_PER_W,), jnp.float32),
            pltpu.VMEM((N_FEAT * TAIL,), jnp.float32),
        ] + [pltpu.SemaphoreType.DMA] * KBUF,
    )
    covn, ccnt, parts, pops = sc(ids, featT, nbf, chan, tailf)

    coverage, covd, means, ns = pl.pallas_call(
        _combine_body,
        out_shape=(
            jax.ShapeDtypeStruct((N_NEIGHBORHOODS,), jnp.float32),
            jax.ShapeDtypeStruct((N_NEIGHBORHOODS,), jnp.int32),
            jax.ShapeDtypeStruct((N_NEIGHBORHOODS, N_FEAT), jnp.float32),
            jax.ShapeDtypeStruct((1, 1), jnp.int32),
        ),
        out_specs=(
            pl.BlockSpec(memory_space=pltpu.VMEM),
            pl.BlockSpec(memory_space=pltpu.VMEM),
            pl.BlockSpec(memory_space=pltpu.VMEM),
            pl.BlockSpec(memory_space=pltpu.SMEM),
        ),
    )(parts.reshape(NW, N_FEAT, N_NEIGHBORHOODS),
      pops.reshape(NW, N_NEIGHBORHOODS), covn, ccnt)

    return coverage, covd.astype(bool), ns[0, 0], means


# speedup vs baseline: 1.6267x; 1.6267x over previous
import jax
import jax.numpy as jnp
from jax import lax
from jax.experimental import pallas as pl
from jax.experimental.pallas import tpu as pltpu
from jax.experimental.pallas import tpu_sc as plsc

N_CHANNELS = 384
N_NEIGHBORHOODS = 512
NHOOD_SIZE = 40
N_SPIKES = 1000000
N_FEAT = 8
MIN_COVERAGE = 0.12

NW = 32
BLK = 128
NBLK = N_SPIKES // BLK
TAIL = N_SPIKES - NBLK * BLK
BPW = NBLK // NW
EXTRA_B = NBLK - BPW * NW
CHUNK_B = 32
N_FULL_CHUNK = BPW // CHUNK_B
REM_B = BPW - N_FULL_CHUNK * CHUNK_B
CHUNK_SP = CHUNK_B * BLK
REM_SP = REM_B * BLK
NB_PER_W = N_NEIGHBORHOODS // NW


def _sc_kernel(ids_hbm, featT_hbm, nbf_hbm, chan_hbm, tailf_hbm,
               covn_hbm, ccnt_hbm, parts_hbm, pops_hbm,
               ids_v, ids_w, feat_v, feat_w, a0, a1, a2, a3, a4, a5, a6, a7,
               pop_v, ind_v, nb_v, chan_v, cov_st, cnt_st, tailf_v,
               sem_i0, sem_i1, sem_f0, sem_f1):
    w = lax.axis_index("c") * 16 + lax.axis_index("s")
    iota = jnp.arange(16, dtype=jnp.int32)
    zeros_f = jnp.zeros((16,), jnp.float32)
    ones_f = jnp.full((16,), 1.0, jnp.float32)
    ones_i = jnp.full((16,), 1, jnp.int32)
    zeros_i = jnp.zeros((16,), jnp.int32)
    accs = [a0, a1, a2, a3, a4, a5, a6, a7]

    ids_bufs = [ids_v, ids_w]
    feat_bufs = [feat_v, feat_w]
    sems_i = [sem_i0, sem_i1]
    sems_f = [sem_f0, sem_f1]
    base_sp = w * (BPW * BLK)
    n_chunks = N_FULL_CHUNK + 1

    def _start(c, buf):
        n_sp = CHUNK_SP if c < N_FULL_CHUNK else REM_SP
        off = base_sp + c * CHUNK_SP
        di = pltpu.async_copy(ids_hbm.at[pl.ds(off, n_sp)],
                              ids_bufs[buf].at[pl.ds(0, n_sp)], sems_i[buf])
        df = pltpu.async_copy(featT_hbm.at[:, pl.ds(off, n_sp)],
                              feat_bufs[buf].at[:, pl.ds(0, n_sp)], sems_f[buf])
        return di, df

    pend = _start(0, 0)

    pltpu.sync_copy(nbf_hbm.at[pl.ds(w * (NB_PER_W * NHOOD_SIZE),
                                     NB_PER_W * NHOOD_SIZE)], nb_v)
    pltpu.sync_copy(chan_hbm, chan_v)

    def _zero_ind(k, carry):
        ind_v[pl.ds(k * 16, 16)] = zeros_f
        return carry
    lax.fori_loop(0, N_CHANNELS // 16, _zero_ind, 0)

    ch0 = chan_v[pl.ds(0, 16)]
    ch1 = chan_v[pl.ds(16, 16)]
    ch2 = chan_v[pl.ds(32, 16)]

    def _nb_body(jj, carry):
        num_vec, cnt_vec = carry
        b = jj * NHOOD_SIZE
        v0 = nb_v[pl.ds(b, 16)]
        v1 = nb_v[pl.ds(b + 16, 16)]
        v2 = nb_v[pl.ds(b + 24, 16)]
        plsc.store_scatter(ind_v, [v0], ones_f)
        plsc.store_scatter(ind_v, [v1], ones_f)
        plsc.store_scatter(ind_v, [v2], ones_f)

        def _sum_ind(k, s):
            return s + ind_v[pl.ds(k * 16, 16)]
        cnt = jnp.sum(lax.fori_loop(0, N_CHANNELS // 16, _sum_ind, zeros_f))
        hits = (plsc.load_gather(ind_v, [ch0])
                + plsc.load_gather(ind_v, [ch1])
                + plsc.load_gather(ind_v, [ch2]))
        here = iota == jj
        num_vec = jnp.where(here, jnp.sum(hits), num_vec)
        cnt_vec = jnp.where(here, cnt, cnt_vec)
        plsc.store_scatter(ind_v, [v0], zeros_f)
        plsc.store_scatter(ind_v, [v1], zeros_f)
        plsc.store_scatter(ind_v, [v2], zeros_f)
        return num_vec, cnt_vec
    num_vec, cnt_vec = lax.fori_loop(
        0, NB_PER_W, _nb_body, (zeros_f, zeros_f))
    cov_st[...] = num_vec
    cnt_st[...] = cnt_vec

    pltpu.sync_copy(cov_st, covn_hbm.at[pl.ds(w * NB_PER_W, NB_PER_W)])
    pltpu.sync_copy(cnt_st, ccnt_hbm.at[pl.ds(w * NB_PER_W, NB_PER_W)])

    def _zero_acc(k, carry):
        for a in accs:
            a[pl.ds(k * 16, 16)] = zeros_f
        pop_v[pl.ds(k * 16, 16)] = zeros_i
        return carry
    lax.fori_loop(0, N_NEIGHBORHOODS // 16, _zero_acc, 0)

    def _process(buf, n_sp):
        ib, fb = ids_bufs[buf], feat_bufs[buf]

        @plsc.parallel_loop(0, n_sp, step=16, unroll=4)
        def _g(s):
            ids16 = ib[pl.ds(s, 16)]
            plsc.addupdate_scatter(pop_v, [ids16], ones_i)
            for j in range(N_FEAT):
                x = fb[j, pl.ds(s, 16)]
                plsc.addupdate_scatter(accs[j], [ids16], x)

    for c in range(n_chunks):
        nxt = _start(c + 1, (c + 1) % 2) if c + 1 < n_chunks else None
        pend[0].wait()
        pend[1].wait()
        _process(c % 2, CHUNK_SP if c < N_FULL_CHUNK else REM_SP)
        pend = nxt

    @pl.when(w < EXTRA_B)
    def _extra():
        off = NW * (BPW * BLK) + w * BLK
        pltpu.sync_copy(ids_hbm.at[pl.ds(off, BLK)], ids_v.at[pl.ds(0, BLK)])
        pltpu.sync_copy(featT_hbm.at[:, pl.ds(off, BLK)],
                        feat_v.at[:, pl.ds(0, BLK)])
        _process(0, BLK)

    @pl.when(w == NW - 1)
    def _tail():
        off = NBLK * BLK
        pltpu.sync_copy(ids_hbm.at[pl.ds(off, TAIL)], ids_v.at[pl.ds(0, TAIL)])
        pltpu.sync_copy(tailf_hbm, tailf_v)

        def _tgroup(t, carry):
            ids16 = ids_v[pl.ds(t * 16, 16)]
            plsc.addupdate_scatter(pop_v, [ids16], ones_i)
            for j in range(N_FEAT):
                x = tailf_v[pl.ds(j * TAIL + t * 16, 16)]
                plsc.addupdate_scatter(accs[j], [ids16], x)
            return carry
        lax.fori_loop(0, TAIL // 16, _tgroup, 0)

    for j in range(N_FEAT):
        pltpu.sync_copy(
            accs[j],
            parts_hbm.at[pl.ds(w * (N_FEAT * N_NEIGHBORHOODS)
                               + j * N_NEIGHBORHOODS, N_NEIGHBORHOODS)])
    pltpu.sync_copy(pop_v, pops_hbm.at[pl.ds(w * N_NEIGHBORHOODS,
                                             N_NEIGHBORHOODS)])


def _combine_body(parts_ref, pops_ref, covn_ref, ccnt_ref,
                  cov_ref, covd_ref, means_ref, ns_ref):
    s = parts_ref[0]
    for wq in range(1, NW):
        s = s + parts_ref[wq]
    pop = jnp.sum(pops_ref[...], axis=0)
    denom = jnp.maximum(pop, 1).astype(jnp.float32)
    means_ref[...] = s.T / denom[:, None]
    coverage = covn_ref[...] / ccnt_ref[...]
    cov_ref[...] = coverage
    covered = coverage >= MIN_COVERAGE
    covd_ref[...] = covered.astype(jnp.int32)
    ns_ref[0, 0] = jnp.sum(jnp.where(covered, pop, 0))


@jax.jit
def kernel(neighborhood_ids, neighborhoods, channels, features):
    ids = neighborhood_ids.astype(jnp.int32)
    nbf = neighborhoods.astype(jnp.int32).reshape(-1)
    chan = channels.astype(jnp.int32)
    featT = features.astype(jnp.float32).T
    tailf = featT[:, NBLK * BLK:].reshape(-1)

    mesh = plsc.VectorSubcoreMesh(core_axis_name="c", subcore_axis_name="s")
    sc = pl.kernel(
        _sc_kernel,
        out_type=(
            jax.ShapeDtypeStruct((N_NEIGHBORHOODS,), jnp.float32),
            jax.ShapeDtypeStruct((N_NEIGHBORHOODS,), jnp.float32),
            jax.ShapeDtypeStruct((NW * N_FEAT * N_NEIGHBORHOODS,), jnp.float32),
            jax.ShapeDtypeStruct((NW * N_NEIGHBORHOODS,), jnp.int32),
        ),
        mesh=mesh,
        compiler_params=pltpu.CompilerParams(needs_layout_passes=False,
                                             use_tc_tiling_on_sc=True),
        scratch_types=[
            pltpu.VMEM((CHUNK_SP,), jnp.int32),
            pltpu.VMEM((CHUNK_SP,), jnp.int32),
            pltpu.VMEM((N_FEAT, CHUNK_SP), jnp.float32),
            pltpu.VMEM((N_FEAT, CHUNK_SP), jnp.float32),
        ] + [pltpu.VMEM((N_NEIGHBORHOODS,), jnp.float32)] * N_FEAT + [
            pltpu.VMEM((N_NEIGHBORHOODS,), jnp.int32),
            pltpu.VMEM((N_CHANNELS,), jnp.float32),
            pltpu.VMEM((NB_PER_W * NHOOD_SIZE,), jnp.int32),
            pltpu.VMEM((48,), jnp.int32),
            pltpu.VMEM((NB_PER_W,), jnp.float32),
            pltpu.VMEM((NB_PER_W,), jnp.float32),
            pltpu.VMEM((N_FEAT * TAIL,), jnp.float32),
            pltpu.SemaphoreType.DMA,
            pltpu.SemaphoreType.DMA,
            pltpu.SemaphoreType.DMA,
            pltpu.SemaphoreType.DMA,
        ],
    )
    covn, ccnt, parts, pops = sc(ids, featT, nbf, chan, tailf)

    coverage, covd, means, ns = pl.pallas_call(
        _combine_body,
        out_shape=(
            jax.ShapeDtypeStruct((N_NEIGHBORHOODS,), jnp.float32),
            jax.ShapeDtypeStruct((N_NEIGHBORHOODS,), jnp.int32),
            jax.ShapeDtypeStruct((N_NEIGHBORHOODS, N_FEAT), jnp.float32),
            jax.ShapeDtypeStruct((1, 1), jnp.int32),
        ),
        out_specs=(
            pl.BlockSpec(memory_space=pltpu.VMEM),
            pl.BlockSpec(memory_space=pltpu.VMEM),
            pl.BlockSpec(memory_space=pltpu.VMEM),
            pl.BlockSpec(memory_space=pltpu.SMEM),
        ),
    )(parts.reshape(NW, N_FEAT, N_NEIGHBORHOODS),
      pops.reshape(NW, N_NEIGHBORHOODS), covn, ccnt)

    return coverage, covd.astype(bool), ns[0, 0], means


# speedup vs baseline: 1.6400x; 1.0082x over previous
import jax
import jax.numpy as jnp
from jax import lax
from jax.experimental import pallas as pl
from jax.experimental.pallas import tpu as pltpu
from jax.experimental.pallas import tpu_sc as plsc

N_CHANNELS = 384
N_NEIGHBORHOODS = 512
NHOOD_SIZE = 40
N_SPIKES = 1000000
N_FEAT = 8
MIN_COVERAGE = 0.12

NW = 32
BLK = 128
NBLK = N_SPIKES // BLK
TAIL = N_SPIKES - NBLK * BLK
BPW = NBLK // NW
EXTRA_B = NBLK - BPW * NW
CHUNK_B = 48
N_FULL_CHUNK = BPW // CHUNK_B
REM_B = BPW - N_FULL_CHUNK * CHUNK_B
CHUNK_SP = CHUNK_B * BLK
REM_SP = REM_B * BLK
NB_PER_W = N_NEIGHBORHOODS // NW


def _sc_kernel(ids_hbm, featT_hbm, nbf_hbm, chan_hbm, tailf_hbm,
               covn_hbm, ccnt_hbm, parts_hbm, pops_hbm,
               ids_v, ids_w, feat_v, feat_w, a0, a1, a2, a3, a4, a5, a6, a7,
               pop_v, ind_v, nb_v, chan_v, cov_st, cnt_st, tailf_v,
               sem_i0, sem_i1, sem_f0, sem_f1):
    w = lax.axis_index("c") * 16 + lax.axis_index("s")
    iota = jnp.arange(16, dtype=jnp.int32)
    zeros_f = jnp.zeros((16,), jnp.float32)
    ones_f = jnp.full((16,), 1.0, jnp.float32)
    ones_i = jnp.full((16,), 1, jnp.int32)
    zeros_i = jnp.zeros((16,), jnp.int32)
    accs = [a0, a1, a2, a3, a4, a5, a6, a7]

    ids_bufs = [ids_v, ids_w]
    feat_bufs = [feat_v, feat_w]
    sems_i = [sem_i0, sem_i1]
    sems_f = [sem_f0, sem_f1]
    base_sp = w * (BPW * BLK)
    n_chunks = N_FULL_CHUNK + 1

    def _start(c, buf):
        n_sp = CHUNK_SP if c < N_FULL_CHUNK else REM_SP
        off = base_sp + c * CHUNK_SP
        di = pltpu.async_copy(ids_hbm.at[pl.ds(off, n_sp)],
                              ids_bufs[buf].at[pl.ds(0, n_sp)], sems_i[buf])
        df = pltpu.async_copy(featT_hbm.at[:, pl.ds(off, n_sp)],
                              feat_bufs[buf].at[:, pl.ds(0, n_sp)], sems_f[buf])
        return di, df

    pend = _start(0, 0)

    pltpu.sync_copy(nbf_hbm.at[pl.ds(w * (NB_PER_W * NHOOD_SIZE),
                                     NB_PER_W * NHOOD_SIZE)], nb_v)
    pltpu.sync_copy(chan_hbm, chan_v)

    def _zero_ind(k, carry):
        ind_v[pl.ds(k * 16, 16)] = zeros_f
        return carry
    lax.fori_loop(0, N_CHANNELS // 16, _zero_ind, 0)

    ch0 = chan_v[pl.ds(0, 16)]
    ch1 = chan_v[pl.ds(16, 16)]
    ch2 = chan_v[pl.ds(32, 16)]

    def _nb_body(jj, carry):
        num_vec, cnt_vec = carry
        b = jj * NHOOD_SIZE
        v0 = nb_v[pl.ds(b, 16)]
        v1 = nb_v[pl.ds(b + 16, 16)]
        v2 = nb_v[pl.ds(b + 24, 16)]
        plsc.store_scatter(ind_v, [v0], ones_f)
        plsc.store_scatter(ind_v, [v1], ones_f)
        plsc.store_scatter(ind_v, [v2], ones_f)

        def _sum_ind(k, s):
            return s + ind_v[pl.ds(k * 16, 16)]
        cnt = jnp.sum(lax.fori_loop(0, N_CHANNELS // 16, _sum_ind, zeros_f))
        hits = (plsc.load_gather(ind_v, [ch0])
                + plsc.load_gather(ind_v, [ch1])
                + plsc.load_gather(ind_v, [ch2]))
        here = iota == jj
        num_vec = jnp.where(here, jnp.sum(hits), num_vec)
        cnt_vec = jnp.where(here, cnt, cnt_vec)
        plsc.store_scatter(ind_v, [v0], zeros_f)
        plsc.store_scatter(ind_v, [v1], zeros_f)
        plsc.store_scatter(ind_v, [v2], zeros_f)
        return num_vec, cnt_vec
    num_vec, cnt_vec = lax.fori_loop(
        0, NB_PER_W, _nb_body, (zeros_f, zeros_f))
    cov_st[...] = num_vec
    cnt_st[...] = cnt_vec

    pltpu.sync_copy(cov_st, covn_hbm.at[pl.ds(w * NB_PER_W, NB_PER_W)])
    pltpu.sync_copy(cnt_st, ccnt_hbm.at[pl.ds(w * NB_PER_W, NB_PER_W)])

    def _zero_acc(k, carry):
        for a in accs:
            a[pl.ds(k * 16, 16)] = zeros_f
        pop_v[pl.ds(k * 16, 16)] = zeros_i
        return carry
    lax.fori_loop(0, N_NEIGHBORHOODS // 16, _zero_acc, 0)

    def _process(buf, n_sp):
        ib, fb = ids_bufs[buf], feat_bufs[buf]

        @plsc.parallel_loop(0, n_sp, step=16, unroll=4)
        def _g(s):
            ids16 = ib[pl.ds(s, 16)]
            plsc.addupdate_scatter(pop_v, [ids16], ones_i)
            for j in range(N_FEAT):
                x = fb[j, pl.ds(s, 16)]
                plsc.addupdate_scatter(accs[j], [ids16], x)

    for c in range(n_chunks):
        nxt = _start(c + 1, (c + 1) % 2) if c + 1 < n_chunks else None
        pend[0].wait()
        pend[1].wait()
        _process(c % 2, CHUNK_SP if c < N_FULL_CHUNK else REM_SP)
        pend = nxt

    @pl.when(w < EXTRA_B)
    def _extra():
        off = NW * (BPW * BLK) + w * BLK
        pltpu.sync_copy(ids_hbm.at[pl.ds(off, BLK)], ids_v.at[pl.ds(0, BLK)])
        pltpu.sync_copy(featT_hbm.at[:, pl.ds(off, BLK)],
                        feat_v.at[:, pl.ds(0, BLK)])
        _process(0, BLK)

    @pl.when(w == NW - 1)
    def _tail():
        off = NBLK * BLK
        pltpu.sync_copy(ids_hbm.at[pl.ds(off, TAIL)], ids_v.at[pl.ds(0, TAIL)])
        pltpu.sync_copy(tailf_hbm, tailf_v)

        def _tgroup(t, carry):
            ids16 = ids_v[pl.ds(t * 16, 16)]
            plsc.addupdate_scatter(pop_v, [ids16], ones_i)
            for j in range(N_FEAT):
                x = tailf_v[pl.ds(j * TAIL + t * 16, 16)]
                plsc.addupdate_scatter(accs[j], [ids16], x)
            return carry
        lax.fori_loop(0, TAIL // 16, _tgroup, 0)

    for j in range(N_FEAT):
        pltpu.sync_copy(
            accs[j],
            parts_hbm.at[pl.ds(w * (N_FEAT * N_NEIGHBORHOODS)
                               + j * N_NEIGHBORHOODS, N_NEIGHBORHOODS)])
    pltpu.sync_copy(pop_v, pops_hbm.at[pl.ds(w * N_NEIGHBORHOODS,
                                             N_NEIGHBORHOODS)])


def _combine_body(parts_ref, pops_ref, covn_ref, ccnt_ref,
                  cov_ref, covd_ref, means_ref, ns_ref):
    s = parts_ref[0]
    for wq in range(1, NW):
        s = s + parts_ref[wq]
    pop = jnp.sum(pops_ref[...], axis=0)
    denom = jnp.maximum(pop, 1).astype(jnp.float32)
    means_ref[...] = s.T / denom[:, None]
    coverage = covn_ref[...] / ccnt_ref[...]
    cov_ref[...] = coverage
    covered = coverage >= MIN_COVERAGE
    covd_ref[...] = covered.astype(jnp.int32)
    ns_ref[0, 0] = jnp.sum(jnp.where(covered, pop, 0))


@jax.jit
def kernel(neighborhood_ids, neighborhoods, channels, features):
    ids = neighborhood_ids.astype(jnp.int32)
    nbf = neighborhoods.astype(jnp.int32).reshape(-1)
    chan = channels.astype(jnp.int32)
    featT = features.astype(jnp.float32).T
    tailf = featT[:, NBLK * BLK:].reshape(-1)

    mesh = plsc.VectorSubcoreMesh(core_axis_name="c", subcore_axis_name="s")
    sc = pl.kernel(
        _sc_kernel,
        out_type=(
            jax.ShapeDtypeStruct((N_NEIGHBORHOODS,), jnp.float32),
            jax.ShapeDtypeStruct((N_NEIGHBORHOODS,), jnp.float32),
            jax.ShapeDtypeStruct((NW * N_FEAT * N_NEIGHBORHOODS,), jnp.float32),
            jax.ShapeDtypeStruct((NW * N_NEIGHBORHOODS,), jnp.int32),
        ),
        mesh=mesh,
        compiler_params=pltpu.CompilerParams(needs_layout_passes=False,
                                             use_tc_tiling_on_sc=True),
        scratch_types=[
            pltpu.VMEM((CHUNK_SP,), jnp.int32),
            pltpu.VMEM((CHUNK_SP,), jnp.int32),
            pltpu.VMEM((N_FEAT, CHUNK_SP), jnp.float32),
            pltpu.VMEM((N_FEAT, CHUNK_SP), jnp.float32),
        ] + [pltpu.VMEM((N_NEIGHBORHOODS,), jnp.float32)] * N_FEAT + [
            pltpu.VMEM((N_NEIGHBORHOODS,), jnp.int32),
            pltpu.VMEM((N_CHANNELS,), jnp.float32),
            pltpu.VMEM((NB_PER_W * NHOOD_SIZE,), jnp.int32),
            pltpu.VMEM((48,), jnp.int32),
            pltpu.VMEM((NB_PER_W,), jnp.float32),
            pltpu.VMEM((NB_PER_W,), jnp.float32),
            pltpu.VMEM((N_FEAT * TAIL,), jnp.float32),
            pltpu.SemaphoreType.DMA,
            pltpu.SemaphoreType.DMA,
            pltpu.SemaphoreType.DMA,
            pltpu.SemaphoreType.DMA,
        ],
    )
    covn, ccnt, parts, pops = sc(ids, featT, nbf, chan, tailf)

    coverage, covd, means, ns = pl.pallas_call(
        _combine_body,
        out_shape=(
            jax.ShapeDtypeStruct((N_NEIGHBORHOODS,), jnp.float32),
            jax.ShapeDtypeStruct((N_NEIGHBORHOODS,), jnp.int32),
            jax.ShapeDtypeStruct((N_NEIGHBORHOODS, N_FEAT), jnp.float32),
            jax.ShapeDtypeStruct((1, 1), jnp.int32),
        ),
        out_specs=(
            pl.BlockSpec(memory_space=pltpu.VMEM),
            pl.BlockSpec(memory_space=pltpu.VMEM),
            pl.BlockSpec(memory_space=pltpu.VMEM),
            pl.BlockSpec(memory_space=pltpu.SMEM),
        ),
    )(parts.reshape(NW, N_FEAT, N_NEIGHBORHOODS),
      pops.reshape(NW, N_NEIGHBORHOODS), covn, ccnt)

    return coverage, covd.astype(bool), ns[0, 0], means


# speedup vs baseline: 1.7238x; 1.0511x over previous
import jax
import jax.numpy as jnp
from jax import lax
from jax.experimental import pallas as pl
from jax.experimental.pallas import tpu as pltpu
from jax.experimental.pallas import tpu_sc as plsc

N_CHANNELS = 384
N_NEIGHBORHOODS = 512
NHOOD_SIZE = 40
N_SPIKES = 1000000
N_FEAT = 8
MIN_COVERAGE = 0.12

NW = 32
BLK = 128
NBLK = N_SPIKES // BLK
TAIL = N_SPIKES - NBLK * BLK
BPW = NBLK // NW
EXTRA_B = NBLK - BPW * NW
CHUNK_B = 48
N_FULL_CHUNK = BPW // CHUNK_B
REM_B = BPW - N_FULL_CHUNK * CHUNK_B
CHUNK_SP = CHUNK_B * BLK
REM_SP = REM_B * BLK
NB_PER_W = N_NEIGHBORHOODS // NW


def _sc_kernel(ids_hbm, featT_hbm, nbf_hbm, chan_hbm, tailf_hbm,
               covn_hbm, ccnt_hbm, parts_hbm, pops_hbm,
               ids_v, ids_w, feat_v, feat_w, a0, a1, a2, a3, a4, a5, a6, a7,
               pop_v, ind_v, nb_v, chan_v, cov_st, cnt_st, tailf_v,
               sem_i0, sem_i1, sem_f0, sem_f1):
    w = lax.axis_index("c") * 16 + lax.axis_index("s")
    iota = jnp.arange(16, dtype=jnp.int32)
    zeros_f = jnp.zeros((16,), jnp.float32)
    ones_f = jnp.full((16,), 1.0, jnp.float32)
    ones_i = jnp.full((16,), 1, jnp.int32)
    zeros_i = jnp.zeros((16,), jnp.int32)
    accs = [a0, a1, a2, a3, a4, a5, a6, a7]

    ids_bufs = [ids_v, ids_w]
    feat_bufs = [feat_v, feat_w]
    sems_i = [sem_i0, sem_i1]
    sems_f = [sem_f0, sem_f1]
    base_sp = w * (BPW * BLK)
    n_chunks = N_FULL_CHUNK + 1

    def _start(c, buf):
        n_sp = CHUNK_SP if c < N_FULL_CHUNK else REM_SP
        off = base_sp + c * CHUNK_SP
        di = pltpu.async_copy(ids_hbm.at[pl.ds(off, n_sp)],
                              ids_bufs[buf].at[pl.ds(0, n_sp)], sems_i[buf])
        df = pltpu.async_copy(featT_hbm.at[:, pl.ds(off, n_sp)],
                              feat_bufs[buf].at[:, pl.ds(0, n_sp)], sems_f[buf])
        return di, df

    pend = _start(0, 0)

    pltpu.sync_copy(nbf_hbm.at[pl.ds(w * (NB_PER_W * NHOOD_SIZE),
                                     NB_PER_W * NHOOD_SIZE)], nb_v)
    pltpu.sync_copy(chan_hbm, chan_v)

    def _zero_ind(k, carry):
        ind_v[pl.ds(k * 16, 16)] = zeros_f
        return carry
    lax.fori_loop(0, N_CHANNELS // 16, _zero_ind, 0)

    ch0 = chan_v[pl.ds(0, 16)]
    ch1 = chan_v[pl.ds(16, 16)]
    ch2 = chan_v[pl.ds(32, 16)]

    def _nb_body(jj, carry):
        num_vec, cnt_vec = carry
        b = jj * NHOOD_SIZE
        v0 = nb_v[pl.ds(b, 16)]
        v1 = nb_v[pl.ds(b + 16, 16)]
        v2 = nb_v[pl.ds(b + 24, 16)]
        plsc.store_scatter(ind_v, [v0], ones_f)
        plsc.store_scatter(ind_v, [v1], ones_f)
        plsc.store_scatter(ind_v, [v2], ones_f)

        def _sum_ind(k, s):
            return s + ind_v[pl.ds(k * 16, 16)]
        cnt = jnp.sum(lax.fori_loop(0, N_CHANNELS // 16, _sum_ind, zeros_f))
        hits = (plsc.load_gather(ind_v, [ch0])
                + plsc.load_gather(ind_v, [ch1])
                + plsc.load_gather(ind_v, [ch2]))
        here = iota == jj
        num_vec = jnp.where(here, jnp.sum(hits), num_vec)
        cnt_vec = jnp.where(here, cnt, cnt_vec)
        plsc.store_scatter(ind_v, [v0], zeros_f)
        plsc.store_scatter(ind_v, [v1], zeros_f)
        plsc.store_scatter(ind_v, [v2], zeros_f)
        return num_vec, cnt_vec
    num_vec, cnt_vec = lax.fori_loop(
        0, NB_PER_W, _nb_body, (zeros_f, zeros_f))
    cov_st[...] = num_vec
    cnt_st[...] = cnt_vec

    pltpu.sync_copy(cov_st, covn_hbm.at[pl.ds(w * NB_PER_W, NB_PER_W)])
    pltpu.sync_copy(cnt_st, ccnt_hbm.at[pl.ds(w * NB_PER_W, NB_PER_W)])

    def _zero_acc(k, carry):
        for a in accs:
            a[pl.ds(k * 16, 16)] = zeros_f
        pop_v[pl.ds(k * 16, 16)] = zeros_i
        return carry
    lax.fori_loop(0, N_NEIGHBORHOODS // 16, _zero_acc, 0)

    def _process(buf, n_sp):
        ib, fb = ids_bufs[buf], feat_bufs[buf]

        @plsc.parallel_loop(0, n_sp, step=16, unroll=4)
        def _g(s):
            ids16 = ib[pl.ds(s, 16)]
            plsc.addupdate_scatter(pop_v, [ids16], ones_i)
            for j in range(N_FEAT):
                x = fb[j, pl.ds(s, 16)]
                plsc.addupdate_scatter(accs[j], [ids16], x)

    for c in range(n_chunks):
        nxt = _start(c + 1, (c + 1) % 2) if c + 1 < n_chunks else None
        pend[0].wait()
        pend[1].wait()
        _process(c % 2, CHUNK_SP if c < N_FULL_CHUNK else REM_SP)
        pend = nxt

    @pl.when(w < EXTRA_B)
    def _extra():
        off = NW * (BPW * BLK) + w * BLK
        pltpu.sync_copy(ids_hbm.at[pl.ds(off, BLK)], ids_v.at[pl.ds(0, BLK)])
        pltpu.sync_copy(featT_hbm.at[:, pl.ds(off, BLK)],
                        feat_v.at[:, pl.ds(0, BLK)])
        _process(0, BLK)

    @pl.when(w == NW - 1)
    def _tail():
        off = NBLK * BLK
        pltpu.sync_copy(ids_hbm.at[pl.ds(off, TAIL)], ids_v.at[pl.ds(0, TAIL)])
        pltpu.sync_copy(tailf_hbm, tailf_v)

        def _tgroup(t, carry):
            ids16 = ids_v[pl.ds(t * 16, 16)]
            plsc.addupdate_scatter(pop_v, [ids16], ones_i)
            for j in range(N_FEAT):
                x = tailf_v[pl.ds(j * TAIL + t * 16, 16)]
                plsc.addupdate_scatter(accs[j], [ids16], x)
            return carry
        lax.fori_loop(0, TAIL // 16, _tgroup, 0)

    for j in range(N_FEAT):
        pltpu.sync_copy(
            accs[j],
            parts_hbm.at[pl.ds(w * (N_FEAT * N_NEIGHBORHOODS)
                               + j * N_NEIGHBORHOODS, N_NEIGHBORHOODS)])
    pltpu.sync_copy(pop_v, pops_hbm.at[pl.ds(w * N_NEIGHBORHOODS,
                                             N_NEIGHBORHOODS)])


def _combine_body(parts_ref, pops_ref, covn_ref, ccnt_ref,
                  cov_ref, covd_ref, means_ref, ns_ref):
    cols = []
    for j in range(N_FEAT):
        sj = parts_ref[pl.ds(j * N_NEIGHBORHOODS, N_NEIGHBORHOODS)]
        for wq in range(1, NW):
            sj = sj + parts_ref[pl.ds((wq * N_FEAT + j) * N_NEIGHBORHOODS,
                                      N_NEIGHBORHOODS)]
        cols.append(sj)
    s = jnp.stack(cols, axis=0)
    pop = pops_ref[pl.ds(0, N_NEIGHBORHOODS)]
    for wq in range(1, NW):
        pop = pop + pops_ref[pl.ds(wq * N_NEIGHBORHOODS, N_NEIGHBORHOODS)]
    denom = jnp.maximum(pop, 1).astype(jnp.float32)
    means_ref[...] = s.T / denom[:, None]
    coverage = covn_ref[...] / ccnt_ref[...]
    cov_ref[...] = coverage
    covered = coverage >= MIN_COVERAGE
    covd_ref[...] = covered.astype(jnp.int32)
    ns_ref[0, 0] = jnp.sum(jnp.where(covered, pop, 0))


@jax.jit
def kernel(neighborhood_ids, neighborhoods, channels, features):
    ids = neighborhood_ids.astype(jnp.int32)
    nbf = neighborhoods.astype(jnp.int32).reshape(-1)
    chan = channels.astype(jnp.int32)
    featT = features.astype(jnp.float32).T
    tailf = featT[:, NBLK * BLK:].reshape(-1)

    mesh = plsc.VectorSubcoreMesh(core_axis_name="c", subcore_axis_name="s")
    sc = pl.kernel(
        _sc_kernel,
        out_type=(
            jax.ShapeDtypeStruct((N_NEIGHBORHOODS,), jnp.float32),
            jax.ShapeDtypeStruct((N_NEIGHBORHOODS,), jnp.float32),
            jax.ShapeDtypeStruct((NW * N_FEAT * N_NEIGHBORHOODS,), jnp.float32),
            jax.ShapeDtypeStruct((NW * N_NEIGHBORHOODS,), jnp.int32),
        ),
        mesh=mesh,
        compiler_params=pltpu.CompilerParams(needs_layout_passes=False,
                                             use_tc_tiling_on_sc=True),
        scratch_types=[
            pltpu.VMEM((CHUNK_SP,), jnp.int32),
            pltpu.VMEM((CHUNK_SP,), jnp.int32),
            pltpu.VMEM((N_FEAT, CHUNK_SP), jnp.float32),
            pltpu.VMEM((N_FEAT, CHUNK_SP), jnp.float32),
        ] + [pltpu.VMEM((N_NEIGHBORHOODS,), jnp.float32)] * N_FEAT + [
            pltpu.VMEM((N_NEIGHBORHOODS,), jnp.int32),
            pltpu.VMEM((N_CHANNELS,), jnp.float32),
            pltpu.VMEM((NB_PER_W * NHOOD_SIZE,), jnp.int32),
            pltpu.VMEM((48,), jnp.int32),
            pltpu.VMEM((NB_PER_W,), jnp.float32),
            pltpu.VMEM((NB_PER_W,), jnp.float32),
            pltpu.VMEM((N_FEAT * TAIL,), jnp.float32),
            pltpu.SemaphoreType.DMA,
            pltpu.SemaphoreType.DMA,
            pltpu.SemaphoreType.DMA,
            pltpu.SemaphoreType.DMA,
        ],
    )
    covn, ccnt, parts, pops = sc(ids, featT, nbf, chan, tailf)

    coverage, covd, means, ns = pl.pallas_call(
        _combine_body,
        out_shape=(
            jax.ShapeDtypeStruct((N_NEIGHBORHOODS,), jnp.float32),
            jax.ShapeDtypeStruct((N_NEIGHBORHOODS,), jnp.int32),
            jax.ShapeDtypeStruct((N_NEIGHBORHOODS, N_FEAT), jnp.float32),
            jax.ShapeDtypeStruct((1, 1), jnp.int32),
        ),
        out_specs=(
            pl.BlockSpec(memory_space=pltpu.VMEM),
            pl.BlockSpec(memory_space=pltpu.VMEM),
            pl.BlockSpec(memory_space=pltpu.VMEM),
            pl.BlockSpec(memory_space=pltpu.SMEM),
        ),
    )(parts, pops, covn, ccnt)

    return coverage, covd.astype(bool), ns[0, 0], means
